# baseline (speedup 1.0000x reference)
import functools

import jax
import jax.numpy as jnp
from jax import lax
from jax.experimental import pallas as pl
from jax.experimental.pallas import tpu as pltpu
from jax.experimental.pallas import tpu_sc as plsc

_NC = 2
_NS = 16
_L = 16
_NW = _NC * _NS

_ROWS = 200
_COLS = 16384
_CPW = _COLS // _NW
_CHUNKS = ((0, 128), (128, 256), (384, 128))
_U = 2


def _sc_remap(xt, table16):
    mesh = plsc.VectorSubcoreMesh(core_axis_name="c", subcore_axis_name="s")

    @functools.partial(
        pl.kernel,
        out_type=jax.ShapeDtypeStruct((_ROWS, _COLS), jnp.int32),
        mesh=mesh,
        scratch_types=(
            [pltpu.VMEM((_ROWS, w), jnp.int32) for _, w in _CHUNKS]
            + [pltpu.VMEM((_L,), jnp.int32)]
            + [pltpu.SemaphoreType.DMA for _ in range(2 * len(_CHUNKS))]
        ),
        compiler_params=pltpu.CompilerParams(needs_layout_passes=False),
    )
    def k(x_hbm, table_hbm, out_hbm, b0, b1, b2, tbuf, *sems):
        bufs = (b0, b1, b2)
        nch = len(_CHUNKS)
        isems, osems = sems[:nch], sems[nch:]
        wid = lax.axis_index("s") * _NC + lax.axis_index("c")
        cbase = wid * _CPW

        ins = [
            pltpu.async_copy(
                x_hbm.at[:, pl.ds(cbase + off, w)], bufs[c], isems[c])
            for c, (off, w) in enumerate(_CHUNKS)
        ]
        outs = []

        pltpu.sync_copy(table_hbm, tbuf)

        tvec = tbuf[...]
        lanes = lax.iota(jnp.int32, _L)
        neg = jnp.int32(-(2**31))

        def lane(j):
            s = jnp.max(jnp.where(lanes == j, tvec, neg))
            return jnp.broadcast_to(s, (_L,))

        t0, t1, t2 = lane(0), lane(1), lane(2)
        tp = t0 | (t1 << 8) | (t2 << 16)
        mask = jnp.full((_L,), 0xFF, jnp.int32)
        three = jnp.full((_L,), 3, jnp.int32)

        for c, (off, w) in enumerate(_CHUNKS):
            buf = bufs[c]
            ins[c].wait()

            @plsc.parallel_loop(0, _ROWS, 1, unroll=_U)
            def _(r):
                for j in range(w // _L):
                    sl = pl.ds(j * _L, _L)
                    xv = buf[r, sl]
                    buf[r, sl] = jnp.right_shift(
                        tp, xv << three) & mask

            outs.append(pltpu.async_copy(
                buf, out_hbm.at[:, pl.ds(cbase + off, w)], osems[c]))

        for o in outs:
            o.wait()

    return k(xt, table16)


def kernel(inputs, lookup_table):
    table16 = jnp.zeros((_L,), jnp.int32).at[:3].set(
        lookup_table.astype(jnp.int32))
    x = inputs if inputs.dtype == jnp.int32 else inputs.astype(jnp.int32)
    out_t = _sc_remap(x.T, table16)
    out = out_t.T
    return out if out.dtype == lookup_table.dtype else out.astype(
        lookup_table.dtype)


# speedup vs baseline: 1.0245x; 1.0245x over previous
import functools

import jax
import jax.numpy as jnp
from jax import lax
from jax.experimental import pallas as pl
from jax.experimental.pallas import tpu as pltpu
from jax.experimental.pallas import tpu_sc as plsc

_NC = 2
_NS = 16
_L = 16
_NW = _NC * _NS

_ROWS = 200
_COLS = 16384
_CPW = _COLS // _NW
_CHC = 256
_NCH = _CPW // _CHC
_NB = 2
_U = 2


def _sc_remap(xt, table16):
    mesh = plsc.VectorSubcoreMesh(core_axis_name="c", subcore_axis_name="s")

    @functools.partial(
        pl.kernel,
        out_type=jax.ShapeDtypeStruct((_ROWS, _COLS), jnp.int32),
        mesh=mesh,
        scratch_types=(
            [pltpu.VMEM((_ROWS, _CHC), jnp.int32) for _ in range(_NB)]
            + [pltpu.VMEM((_L,), jnp.int32)]
            + [pltpu.SemaphoreType.DMA for _ in range(2 * _NB)]
        ),
        compiler_params=pltpu.CompilerParams(needs_layout_passes=False),
    )
    def k(x_hbm, table_hbm, out_hbm, b0, b1, tbuf, *sems):
        bufs = (b0, b1)
        isems, osems = sems[:_NB], sems[_NB:]
        wid = lax.axis_index("s") * _NC + lax.axis_index("c")
        cbase = wid * _CPW

        def start_in(c):
            return pltpu.async_copy(
                x_hbm.at[:, pl.ds(cbase + c * _CHC, _CHC)], bufs[c % _NB],
                isems[c % _NB])

        ins = {0: start_in(0)}
        outs = {}

        pltpu.sync_copy(table_hbm, tbuf)

        tvec = tbuf[...]
        lanes = lax.iota(jnp.int32, _L)
        neg = jnp.int32(-(2**31))

        def lane(j):
            s = jnp.max(jnp.where(lanes == j, tvec, neg))
            return jnp.broadcast_to(s, (_L,))

        t0, t1, t2 = lane(0), lane(1), lane(2)
        tp = t0 | (t1 << 8) | (t2 << 16)
        mask = jnp.full((_L,), 0xFF, jnp.int32)
        three = jnp.full((_L,), 3, jnp.int32)

        for c in range(_NCH):
            buf = bufs[c % _NB]
            if c + 1 < _NCH:
                if c + 1 >= _NB:
                    outs[c + 1 - _NB].wait()
                ins[c + 1] = start_in(c + 1)
            ins[c].wait()

            @plsc.parallel_loop(0, _ROWS, 1, unroll=_U)
            def _(r):
                for j in range(_CHC // _L):
                    sl = pl.ds(j * _L, _L)
                    xv = buf[r, sl]
                    buf[r, sl] = jnp.right_shift(
                        tp, xv << three) & mask

            outs[c] = pltpu.async_copy(
                buf, out_hbm.at[:, pl.ds(cbase + c * _CHC, _CHC)],
                osems[c % _NB])

        for c in range(max(0, _NCH - _NB), _NCH):
            outs[c].wait()

    return k(xt, table16)


def kernel(inputs, lookup_table):
    table16 = jnp.zeros((_L,), jnp.int32).at[:3].set(
        lookup_table.astype(jnp.int32))
    x = inputs if inputs.dtype == jnp.int32 else inputs.astype(jnp.int32)
    out_t = _sc_remap(x.T, table16)
    out = out_t.T
    return out if out.dtype == lookup_table.dtype else out.astype(
        lookup_table.dtype)
